# baseline (speedup 1.0000x reference)
import functools

import jax
import jax.numpy as jnp
from jax import lax
from jax.experimental import pallas as pl
from jax.experimental.pallas import tpu as pltpu
from jax.experimental.pallas import tpu_sc as plsc

N = 10000
D_IN = 128
H = 32
WIDE = 48
DUMMY = N
NP = 10240
BM = 512
NTILES = 32
BLK = 128
NBLK = 82
NBLK_ALL = NBLK + 2
EPT = NBLK * BLK
E_PAD = NTILES * EPT
ROWS_PER_TILE = NP // 16

_PREC = jax.lax.Precision.HIGHEST


def _tc_first_body(x_ref, m_ref, wsd_ref, h_ref, sdt_ref):
    xb = x_ref[...]
    ones_col = (jax.lax.broadcasted_iota(jnp.int32, (1, WIDE), 1) == H)
    h_ref[...] = (jnp.dot(xb, m_ref[...], precision=_PREC,
                          preferred_element_type=jnp.float32)
                  + ones_col.astype(jnp.float32))
    sdt_ref[...] = jax.lax.dot_general(
        wsd_ref[...], xb, (((1,), (1,)), ((), ())),
        precision=_PREC, preferred_element_type=jnp.float32)


def _tc_mid_body(acc_ref, b_ref, m_ref, wsd_ref, h_ref, sdt_ref):
    a = acc_ref[...]
    num = a[0, :, :H] + a[1, :, :H]
    den = a[0, :, H:H + 1] + a[1, :, H:H + 1]
    hprev = jnp.maximum(num / (den + 1e-16) + b_ref[...], 0.0)
    ones_col = (jax.lax.broadcasted_iota(jnp.int32, (1, WIDE), 1) == H)
    h_ref[...] = (jnp.dot(hprev, m_ref[...], precision=_PREC,
                          preferred_element_type=jnp.float32)
                  + ones_col.astype(jnp.float32))
    sdt_ref[...] = jax.lax.dot_general(
        wsd_ref[...], hprev, (((1,), (1,)), ((), ())),
        precision=_PREC, preferred_element_type=jnp.float32)


def _tc_final_body(acc_ref, b_ref, wf_ref, bf_ref, out_ref):
    a = acc_ref[...]
    num = a[0, :, :H] + a[1, :, :H]
    den = a[0, :, H:H + 1] + a[1, :, H:H + 1]
    hprev = jnp.maximum(num / (den + 1e-16) + b_ref[...], 0.0)
    out_ref[...] = (jnp.dot(hprev, wf_ref[...], precision=_PREC,
                            preferred_element_type=jnp.float32)
                    + bf_ref[...])


_G = NP // BM


def _tc_first(x_pad, m1, wsd1):
    return pl.pallas_call(
        _tc_first_body,
        grid=(_G,),
        in_specs=[
            pl.BlockSpec((BM, D_IN), lambda i: (i, 0)),
            pl.BlockSpec((D_IN, WIDE), lambda i: (0, 0)),
            pl.BlockSpec((8, D_IN), lambda i: (0, 0)),
        ],
        out_specs=[
            pl.BlockSpec((BM, WIDE), lambda i: (i, 0)),
            pl.BlockSpec((8, BM), lambda i: (0, i)),
        ],
        out_shape=[
            jax.ShapeDtypeStruct((NP, WIDE), jnp.float32),
            jax.ShapeDtypeStruct((8, NP), jnp.float32),
        ],
    )(x_pad, m1, wsd1)


def _tc_mid(acc, bvec, m2, wsd2):
    return pl.pallas_call(
        _tc_mid_body,
        grid=(_G,),
        in_specs=[
            pl.BlockSpec((2, BM, WIDE), lambda i: (0, i, 0)),
            pl.BlockSpec((1, H), lambda i: (0, 0)),
            pl.BlockSpec((H, WIDE), lambda i: (0, 0)),
            pl.BlockSpec((8, H), lambda i: (0, 0)),
        ],
        out_specs=[
            pl.BlockSpec((BM, WIDE), lambda i: (i, 0)),
            pl.BlockSpec((8, BM), lambda i: (0, i)),
        ],
        out_shape=[
            jax.ShapeDtypeStruct((NP, WIDE), jnp.float32),
            jax.ShapeDtypeStruct((8, NP), jnp.float32),
        ],
    )(acc, bvec, m2, wsd2)


def _tc_final(acc, bvec, wf, bf):
    return pl.pallas_call(
        _tc_final_body,
        grid=(_G,),
        in_specs=[
            pl.BlockSpec((2, BM, WIDE), lambda i: (0, i, 0)),
            pl.BlockSpec((1, H), lambda i: (0, 0)),
            pl.BlockSpec((H, H), lambda i: (0, 0)),
            pl.BlockSpec((1, H), lambda i: (0, 0)),
        ],
        out_specs=pl.BlockSpec((BM, H), lambda i: (i, 0)),
        out_shape=jax.ShapeDtypeStruct((NP, H), jnp.float32),
    )(acc, bvec, wf, bf)


def _edge_pass_body(h_hbm, sdt_hbm, src_hbm, dst_hbm, zeros_hbm, out_hbm,
                    src_v, dst_v, s_v, d_v, msg_a, msg_b, ex_v, acc_sh,
                    gsem_a, gsem_b, ssem_a, ssem_b):
    core = lax.axis_index("c")
    sub = lax.axis_index("s")
    wid = sub * 2 + core

    pltpu.sync_copy(src_hbm.at[wid], src_v)
    pltpu.sync_copy(dst_hbm.at[wid], dst_v)
    pltpu.sync_copy(sdt_hbm.at[0], s_v)
    pltpu.sync_copy(sdt_hbm.at[1], d_v)
    pltpu.sync_copy(zeros_hbm, acc_sh.at[pl.ds(sub * ROWS_PER_TILE,
                                               ROWS_PER_TILE)])
    plsc.subcore_barrier()

    def alpha(b):
        for g in range(BLK // 16):
            src16 = src_v[b, pl.ds(g * 16, 16)]
            dst16 = dst_v[b, pl.ds(g * 16, 16)]
            al = plsc.load_gather(s_v, [src16]) + plsc.load_gather(d_v, [dst16])
            al = jnp.maximum(al, 0.2 * al)
            ex_v[pl.ds(16 + g * 16, 16)] = jnp.exp(al)

    def scale(msg_v):
        for j in range(BLK):
            e = plsc.load_gather(ex_v, [jnp.full((16,), 16 + j, jnp.int32)])
            msg_v[j, pl.ds(0, 16)] = msg_v[j, pl.ds(0, 16)] * e
            msg_v[j, pl.ds(16, 16)] = msg_v[j, pl.ds(16, 16)] * e
            msg_v[j, pl.ds(32, 16)] = e

    def gather(b, msg_v, sem):
        return pltpu.async_copy(h_hbm.at[src_v.at[b]], msg_v, sem)

    def gwait(msg_v, sem):
        pltpu.make_async_copy(h_hbm.at[src_v.at[0]], msg_v, sem).wait()

    gather(0, msg_a, gsem_a)
    gather(1, msg_b, gsem_b)

    def body(i, carry):
        b0 = 2 * i
        b1 = 2 * i + 1
        alpha(b0)
        gwait(msg_a, gsem_a)
        scale(msg_a)
        sca = pltpu.async_copy(msg_a, acc_sh.at[dst_v.at[b0]], ssem_a,
                               add=True)
        alpha(b1)
        gwait(msg_b, gsem_b)
        scale(msg_b)
        scb = pltpu.async_copy(msg_b, acc_sh.at[dst_v.at[b1]], ssem_b,
                               add=True)
        sca.wait()
        gather(b0 + 2, msg_a, gsem_a)
        scb.wait()
        gather(b1 + 2, msg_b, gsem_b)
        return carry

    lax.fori_loop(0, NBLK // 2, body, 0)

    pltpu.make_async_copy(h_hbm.at[src_v.at[NBLK]], msg_a, gsem_a).wait()
    pltpu.make_async_copy(h_hbm.at[src_v.at[NBLK + 1]], msg_b, gsem_b).wait()

    plsc.subcore_barrier()
    row0 = sub * ROWS_PER_TILE
    pltpu.sync_copy(acc_sh.at[pl.ds(row0, ROWS_PER_TILE)],
                    out_hbm.at[core, pl.ds(row0, ROWS_PER_TILE)])


_edge_pass = functools.partial(
    pl.kernel,
    out_type=jax.ShapeDtypeStruct((2, NP, WIDE), jnp.float32),
    mesh=plsc.VectorSubcoreMesh(core_axis_name="c", subcore_axis_name="s"),
    compiler_params=pltpu.CompilerParams(needs_layout_passes=False,
                                         use_tc_tiling_on_sc=False),
    scratch_types=[
        pltpu.VMEM((NBLK_ALL, BLK), jnp.int32),
        pltpu.VMEM((NBLK_ALL, BLK), jnp.int32),
        pltpu.VMEM((NP,), jnp.float32),
        pltpu.VMEM((NP,), jnp.float32),
        pltpu.VMEM((BLK, WIDE), jnp.float32),
        pltpu.VMEM((BLK, WIDE), jnp.float32),
        pltpu.VMEM((BLK + 16,), jnp.float32),
        pltpu.VMEM_SHARED((NP, WIDE), jnp.float32),
        pltpu.SemaphoreType.DMA,
        pltpu.SemaphoreType.DMA,
        pltpu.SemaphoreType.DMA,
        pltpu.SemaphoreType.DMA,
    ],
)(_edge_pass_body)


def _widen(w, a_src, a_dst):
    k = w.shape[0]
    m = jnp.zeros((k, WIDE), jnp.float32).at[:, :H].set(w)
    wsd = (jnp.zeros((8, k), jnp.float32)
           .at[0].set(jnp.dot(w, a_src, precision=_PREC))
           .at[1].set(jnp.dot(w, a_dst, precision=_PREC)))
    return m, wsd


def kernel(x, edge_index, W1, a_src1, a_dst1, b1, W2, a_src2, a_dst2, b2,
           Wf, bf):
    m1, wsd1 = _widen(W1, a_src1, a_dst1)
    m2, wsd2 = _widen(W2, a_src2, a_dst2)
    b1r = b1.reshape(1, H)
    b2r = b2.reshape(1, H)
    bfr = bf.reshape(1, H)

    loops = jnp.arange(N, dtype=jnp.int32)
    npad = E_PAD - (edge_index.shape[1] + N)
    src = jnp.concatenate([edge_index[0].astype(jnp.int32), loops,
                           jnp.zeros((npad,), jnp.int32)])
    dst = jnp.concatenate([edge_index[1].astype(jnp.int32), loops,
                           jnp.full((npad,), DUMMY, jnp.int32)])
    src3 = jnp.concatenate(
        [src.reshape(NTILES, NBLK, BLK),
         jnp.zeros((NTILES, 2, BLK), jnp.int32)], axis=1)
    dst3 = jnp.concatenate(
        [dst.reshape(NTILES, NBLK, BLK),
         jnp.full((NTILES, 2, BLK), DUMMY, jnp.int32)], axis=1)

    x_pad = jnp.zeros((NP, D_IN), jnp.float32).at[:N].set(x)
    zeros_blk = jnp.zeros((ROWS_PER_TILE, WIDE), jnp.float32)

    h1, sdt1 = _tc_first(x_pad, m1, wsd1)
    acc1 = _edge_pass(h1, sdt1, src3, dst3, zeros_blk)
    h2, sdt2 = _tc_mid(acc1, b1r, m2, wsd2)
    acc2 = _edge_pass(h2, sdt2, src3, dst3, zeros_blk)
    out = _tc_final(acc2, b2r, Wf, bfr)
    return out[:N]


# speedup vs baseline: 1.0815x; 1.0815x over previous
import functools

import jax
import jax.numpy as jnp
from jax import lax
from jax.experimental import pallas as pl
from jax.experimental.pallas import tpu as pltpu
from jax.experimental.pallas import tpu_sc as plsc

N = 10000
D_IN = 128
H = 32
WIDE = 48
DUMMY = N
NP = 10240
BM = 512
NTILES = 32
BLK = 128
NBLK = 82
NBLK_ALL = NBLK + 2
EPT = NBLK * BLK
E_PAD = NTILES * EPT
ROWS_PER_TILE = NP // 16

_PREC = jax.lax.Precision.HIGHEST


def _tc_first_body(x_ref, m_ref, wsd_ref, h_ref, sdt_ref):
    xb = x_ref[...]
    ones_col = (jax.lax.broadcasted_iota(jnp.int32, (1, WIDE), 1) == H)
    h_ref[...] = (jnp.dot(xb, m_ref[...], precision=_PREC,
                          preferred_element_type=jnp.float32)
                  + ones_col.astype(jnp.float32))
    sdt_ref[...] = jax.lax.dot_general(
        wsd_ref[...], xb, (((1,), (1,)), ((), ())),
        precision=_PREC, preferred_element_type=jnp.float32)


def _tc_mid_body(acc_ref, b_ref, m_ref, wsd_ref, h_ref, sdt_ref):
    a = acc_ref[...]
    num = a[0, :, :H] + a[1, :, :H]
    den = a[0, :, H:H + 1] + a[1, :, H:H + 1]
    hprev = jnp.maximum(num / (den + 1e-16) + b_ref[...], 0.0)
    ones_col = (jax.lax.broadcasted_iota(jnp.int32, (1, WIDE), 1) == H)
    h_ref[...] = (jnp.dot(hprev, m_ref[...], precision=_PREC,
                          preferred_element_type=jnp.float32)
                  + ones_col.astype(jnp.float32))
    sdt_ref[...] = jax.lax.dot_general(
        wsd_ref[...], hprev, (((1,), (1,)), ((), ())),
        precision=_PREC, preferred_element_type=jnp.float32)


def _tc_final_body(acc_ref, b_ref, wf_ref, bf_ref, out_ref):
    a = acc_ref[...]
    num = a[0, :, :H] + a[1, :, :H]
    den = a[0, :, H:H + 1] + a[1, :, H:H + 1]
    hprev = jnp.maximum(num / (den + 1e-16) + b_ref[...], 0.0)
    out_ref[...] = (jnp.dot(hprev, wf_ref[...], precision=_PREC,
                            preferred_element_type=jnp.float32)
                    + bf_ref[...])


_G = NP // BM


def _tc_first(x_pad, m1, wsd1):
    return pl.pallas_call(
        _tc_first_body,
        grid=(_G,),
        in_specs=[
            pl.BlockSpec((BM, D_IN), lambda i: (i, 0)),
            pl.BlockSpec((D_IN, WIDE), lambda i: (0, 0)),
            pl.BlockSpec((8, D_IN), lambda i: (0, 0)),
        ],
        out_specs=[
            pl.BlockSpec((BM, WIDE), lambda i: (i, 0)),
            pl.BlockSpec((8, BM), lambda i: (0, i)),
        ],
        out_shape=[
            jax.ShapeDtypeStruct((NP, WIDE), jnp.float32),
            jax.ShapeDtypeStruct((8, NP), jnp.float32),
        ],
    )(x_pad, m1, wsd1)


def _tc_mid(acc, bvec, m2, wsd2):
    return pl.pallas_call(
        _tc_mid_body,
        grid=(_G,),
        in_specs=[
            pl.BlockSpec((2, BM, WIDE), lambda i: (0, i, 0)),
            pl.BlockSpec((1, H), lambda i: (0, 0)),
            pl.BlockSpec((H, WIDE), lambda i: (0, 0)),
            pl.BlockSpec((8, H), lambda i: (0, 0)),
        ],
        out_specs=[
            pl.BlockSpec((BM, WIDE), lambda i: (i, 0)),
            pl.BlockSpec((8, BM), lambda i: (0, i)),
        ],
        out_shape=[
            jax.ShapeDtypeStruct((NP, WIDE), jnp.float32),
            jax.ShapeDtypeStruct((8, NP), jnp.float32),
        ],
    )(acc, bvec, m2, wsd2)


def _tc_final(acc, bvec, wf, bf):
    return pl.pallas_call(
        _tc_final_body,
        grid=(_G,),
        in_specs=[
            pl.BlockSpec((2, BM, WIDE), lambda i: (0, i, 0)),
            pl.BlockSpec((1, H), lambda i: (0, 0)),
            pl.BlockSpec((H, H), lambda i: (0, 0)),
            pl.BlockSpec((1, H), lambda i: (0, 0)),
        ],
        out_specs=pl.BlockSpec((BM, H), lambda i: (i, 0)),
        out_shape=jax.ShapeDtypeStruct((NP, H), jnp.float32),
    )(acc, bvec, wf, bf)


def _edge_pass_body(h_hbm, sdt_hbm, src_hbm, dst_hbm, zeros_hbm, out_hbm,
                    src_v, dst_v, s_v, d_v, msg_a, msg_b, ex_v, acc_sh,
                    gsem_a, gsem_b, ssem_a, ssem_b):
    core = lax.axis_index("c")
    sub = lax.axis_index("s")
    wid = sub * 2 + core

    pltpu.sync_copy(src_hbm.at[wid], src_v)
    pltpu.sync_copy(dst_hbm.at[wid], dst_v)
    pltpu.sync_copy(sdt_hbm.at[0], s_v)
    pltpu.sync_copy(sdt_hbm.at[1], d_v)
    pltpu.sync_copy(zeros_hbm, acc_sh.at[pl.ds(sub * ROWS_PER_TILE,
                                               ROWS_PER_TILE)])
    plsc.subcore_barrier()

    def alpha(b):
        for g in range(BLK // 16):
            src16 = src_v[b, pl.ds(g * 16, 16)]
            dst16 = dst_v[b, pl.ds(g * 16, 16)]
            al = plsc.load_gather(s_v, [src16]) + plsc.load_gather(d_v, [dst16])
            al = jnp.maximum(al, 0.2 * al)
            ex_v[pl.ds(16 + g * 16, 16)] = jnp.exp(al)

    def scale(msg_v):
        for j in range(BLK):
            e = plsc.load_gather(ex_v, [jnp.full((16,), 16 + j, jnp.int32)])
            msg_v[j, pl.ds(0, 16)] = msg_v[j, pl.ds(0, 16)] * e
            msg_v[j, pl.ds(16, 16)] = msg_v[j, pl.ds(16, 16)] * e
            msg_v[j, pl.ds(32, 16)] = e

    def gather(b, msg_v, sem):
        return pltpu.async_copy(h_hbm.at[src_v.at[b]], msg_v, sem)

    def gwait(msg_v, sem):
        pltpu.make_async_copy(h_hbm.at[src_v.at[0]], msg_v, sem).wait()

    gather(0, msg_a, gsem_a)
    gather(1, msg_b, gsem_b)

    def body(i, carry):
        b0 = 2 * i
        b1 = 2 * i + 1
        alpha(b0)
        gwait(msg_a, gsem_a)
        scale(msg_a)
        pltpu.sync_copy(msg_a, acc_sh.at[dst_v.at[b0]], add=True)
        gather(b0 + 2, msg_a, gsem_a)
        alpha(b1)
        gwait(msg_b, gsem_b)
        scale(msg_b)
        pltpu.sync_copy(msg_b, acc_sh.at[dst_v.at[b1]], add=True)
        gather(b1 + 2, msg_b, gsem_b)
        return carry

    lax.fori_loop(0, NBLK // 2, body, 0)

    pltpu.make_async_copy(h_hbm.at[src_v.at[NBLK]], msg_a, gsem_a).wait()
    pltpu.make_async_copy(h_hbm.at[src_v.at[NBLK + 1]], msg_b, gsem_b).wait()

    plsc.subcore_barrier()
    row0 = sub * ROWS_PER_TILE
    pltpu.sync_copy(acc_sh.at[pl.ds(row0, ROWS_PER_TILE)],
                    out_hbm.at[core, pl.ds(row0, ROWS_PER_TILE)])


_edge_pass = functools.partial(
    pl.kernel,
    out_type=jax.ShapeDtypeStruct((2, NP, WIDE), jnp.float32),
    mesh=plsc.VectorSubcoreMesh(core_axis_name="c", subcore_axis_name="s"),
    compiler_params=pltpu.CompilerParams(needs_layout_passes=False,
                                         use_tc_tiling_on_sc=False),
    scratch_types=[
        pltpu.VMEM((NBLK_ALL, BLK), jnp.int32),
        pltpu.VMEM((NBLK_ALL, BLK), jnp.int32),
        pltpu.VMEM((NP,), jnp.float32),
        pltpu.VMEM((NP,), jnp.float32),
        pltpu.VMEM((BLK, WIDE), jnp.float32),
        pltpu.VMEM((BLK, WIDE), jnp.float32),
        pltpu.VMEM((BLK + 16,), jnp.float32),
        pltpu.VMEM_SHARED((NP, WIDE), jnp.float32),
        pltpu.SemaphoreType.DMA,
        pltpu.SemaphoreType.DMA,
        pltpu.SemaphoreType.DMA,
        pltpu.SemaphoreType.DMA,
    ],
)(_edge_pass_body)


def _widen(w, a_src, a_dst):
    k = w.shape[0]
    m = jnp.zeros((k, WIDE), jnp.float32).at[:, :H].set(w)
    wsd = (jnp.zeros((8, k), jnp.float32)
           .at[0].set(jnp.dot(w, a_src, precision=_PREC))
           .at[1].set(jnp.dot(w, a_dst, precision=_PREC)))
    return m, wsd


def kernel(x, edge_index, W1, a_src1, a_dst1, b1, W2, a_src2, a_dst2, b2,
           Wf, bf):
    m1, wsd1 = _widen(W1, a_src1, a_dst1)
    m2, wsd2 = _widen(W2, a_src2, a_dst2)
    b1r = b1.reshape(1, H)
    b2r = b2.reshape(1, H)
    bfr = bf.reshape(1, H)

    loops = jnp.arange(N, dtype=jnp.int32)
    npad = E_PAD - (edge_index.shape[1] + N)
    src = jnp.concatenate([edge_index[0].astype(jnp.int32), loops,
                           jnp.zeros((npad,), jnp.int32)])
    dst = jnp.concatenate([edge_index[1].astype(jnp.int32), loops,
                           jnp.full((npad,), DUMMY, jnp.int32)])
    src3 = jnp.concatenate(
        [src.reshape(NTILES, NBLK, BLK),
         jnp.zeros((NTILES, 2, BLK), jnp.int32)], axis=1)
    dst3 = jnp.concatenate(
        [dst.reshape(NTILES, NBLK, BLK),
         jnp.full((NTILES, 2, BLK), DUMMY, jnp.int32)], axis=1)

    x_pad = jnp.zeros((NP, D_IN), jnp.float32).at[:N].set(x)
    zeros_blk = jnp.zeros((ROWS_PER_TILE, WIDE), jnp.float32)

    h1, sdt1 = _tc_first(x_pad, m1, wsd1)
    acc1 = _edge_pass(h1, sdt1, src3, dst3, zeros_blk)
    h2, sdt2 = _tc_mid(acc1, b1r, m2, wsd2)
    acc2 = _edge_pass(h2, sdt2, src3, dst3, zeros_blk)
    out = _tc_final(acc2, b2r, Wf, bfr)
    return out[:N]


# speedup vs baseline: 1.2269x; 1.1344x over previous
import functools

import jax
import jax.numpy as jnp
from jax import lax
from jax.experimental import pallas as pl
from jax.experimental.pallas import tpu as pltpu
from jax.experimental.pallas import tpu_sc as plsc

N = 10000
D_IN = 128
H = 32
WIDE = 48
DUMMY = N
NP = 10240
BM = 512
NTILES = 32
BLK = 128
NBLK = 82
NBLK_ALL = NBLK + 2
EPT = NBLK * BLK
E_PAD = NTILES * EPT
ROWS_PER_TILE = NP // 16

_PREC = jax.lax.Precision.HIGHEST


def _tc_first_body(x_ref, m_ref, wsd_ref, h_ref, sdt_ref):
    xb = x_ref[...]
    ones_col = (jax.lax.broadcasted_iota(jnp.int32, (1, WIDE), 1) == H)
    h_ref[...] = (jnp.dot(xb, m_ref[...], precision=_PREC,
                          preferred_element_type=jnp.float32)
                  + ones_col.astype(jnp.float32))
    sdt_ref[...] = jax.lax.dot_general(
        wsd_ref[...], xb, (((1,), (1,)), ((), ())),
        precision=_PREC, preferred_element_type=jnp.float32)


def _tc_mid_body(acc_ref, b_ref, m_ref, wsd_ref, h_ref, sdt_ref):
    a = acc_ref[...]
    num = a[0, :, :H] + a[1, :, :H]
    den = a[0, :, H:H + 1] + a[1, :, H:H + 1]
    hprev = jnp.maximum(num / (den + 1e-16) + b_ref[...], 0.0)
    ones_col = (jax.lax.broadcasted_iota(jnp.int32, (1, WIDE), 1) == H)
    h_ref[...] = (jnp.dot(hprev, m_ref[...], precision=_PREC,
                          preferred_element_type=jnp.float32)
                  + ones_col.astype(jnp.float32))
    sdt_ref[...] = jax.lax.dot_general(
        wsd_ref[...], hprev, (((1,), (1,)), ((), ())),
        precision=_PREC, preferred_element_type=jnp.float32)


def _tc_final_body(acc_ref, b_ref, wf_ref, bf_ref, out_ref):
    a = acc_ref[...]
    num = a[0, :, :H] + a[1, :, :H]
    den = a[0, :, H:H + 1] + a[1, :, H:H + 1]
    hprev = jnp.maximum(num / (den + 1e-16) + b_ref[...], 0.0)
    out_ref[...] = (jnp.dot(hprev, wf_ref[...], precision=_PREC,
                            preferred_element_type=jnp.float32)
                    + bf_ref[...])


_G = NP // BM


def _tc_first(x_pad, m1, wsd1):
    return pl.pallas_call(
        _tc_first_body,
        grid=(_G,),
        in_specs=[
            pl.BlockSpec((BM, D_IN), lambda i: (i, 0)),
            pl.BlockSpec((D_IN, WIDE), lambda i: (0, 0)),
            pl.BlockSpec((8, D_IN), lambda i: (0, 0)),
        ],
        out_specs=[
            pl.BlockSpec((BM, WIDE), lambda i: (i, 0)),
            pl.BlockSpec((8, BM), lambda i: (0, i)),
        ],
        out_shape=[
            jax.ShapeDtypeStruct((NP, WIDE), jnp.float32),
            jax.ShapeDtypeStruct((8, NP), jnp.float32),
        ],
    )(x_pad, m1, wsd1)


def _tc_mid(acc, bvec, m2, wsd2):
    return pl.pallas_call(
        _tc_mid_body,
        grid=(_G,),
        in_specs=[
            pl.BlockSpec((2, BM, WIDE), lambda i: (0, i, 0)),
            pl.BlockSpec((1, H), lambda i: (0, 0)),
            pl.BlockSpec((H, WIDE), lambda i: (0, 0)),
            pl.BlockSpec((8, H), lambda i: (0, 0)),
        ],
        out_specs=[
            pl.BlockSpec((BM, WIDE), lambda i: (i, 0)),
            pl.BlockSpec((8, BM), lambda i: (0, i)),
        ],
        out_shape=[
            jax.ShapeDtypeStruct((NP, WIDE), jnp.float32),
            jax.ShapeDtypeStruct((8, NP), jnp.float32),
        ],
    )(acc, bvec, m2, wsd2)


def _tc_final(acc, bvec, wf, bf):
    return pl.pallas_call(
        _tc_final_body,
        grid=(_G,),
        in_specs=[
            pl.BlockSpec((2, BM, WIDE), lambda i: (0, i, 0)),
            pl.BlockSpec((1, H), lambda i: (0, 0)),
            pl.BlockSpec((H, H), lambda i: (0, 0)),
            pl.BlockSpec((1, H), lambda i: (0, 0)),
        ],
        out_specs=pl.BlockSpec((BM, H), lambda i: (i, 0)),
        out_shape=jax.ShapeDtypeStruct((NP, H), jnp.float32),
    )(acc, bvec, wf, bf)


def _edge_pass_body(h_hbm, sdt_hbm, src_hbm, dst_hbm, zeros_hbm, out_hbm,
                    src_v, dst_v, s_v, d_v, msg_a, msg_b, ex_v, acc_sh,
                    gsem_a, gsem_b, ssem_a, ssem_b):
    core = lax.axis_index("c")
    sub = lax.axis_index("s")
    wid = sub * 2 + core

    pltpu.sync_copy(src_hbm.at[wid], src_v)
    pltpu.sync_copy(dst_hbm.at[wid], dst_v)
    pltpu.sync_copy(sdt_hbm.at[0], s_v)
    pltpu.sync_copy(sdt_hbm.at[1], d_v)
    pltpu.sync_copy(zeros_hbm, acc_sh.at[pl.ds(sub * ROWS_PER_TILE,
                                               ROWS_PER_TILE)])
    plsc.subcore_barrier()

    def alpha(b):
        for g in range(BLK // 16):
            src16 = src_v[b, pl.ds(g * 16, 16)]
            dst16 = dst_v[b, pl.ds(g * 16, 16)]
            al = plsc.load_gather(s_v, [src16]) + plsc.load_gather(d_v, [dst16])
            al = jnp.maximum(al, 0.2 * al)
            ex_v[pl.ds(16 + g * 16, 16)] = jnp.exp(al)

    def scale(msg_v):
        for j in range(BLK):
            e = plsc.load_gather(ex_v, [jnp.full((16,), 16 + j, jnp.int32)])
            msg_v[j, pl.ds(0, 16)] = msg_v[j, pl.ds(0, 16)] * e
            msg_v[j, pl.ds(16, 16)] = msg_v[j, pl.ds(16, 16)] * e
            msg_v[j, pl.ds(32, 16)] = e

    def gather(b, msg_v, sem):
        return pltpu.async_copy(h_hbm.at[src_v.at[b]], msg_v, sem)

    def body(b, carry):
        cp = gather(b, msg_a, gsem_a)
        alpha(b)
        cp.wait()
        scale(msg_a)
        pltpu.sync_copy(msg_a, acc_sh.at[dst_v.at[b]], add=True)
        return carry

    lax.fori_loop(0, NBLK, body, 0)

    plsc.subcore_barrier()
    row0 = sub * ROWS_PER_TILE
    pltpu.sync_copy(acc_sh.at[pl.ds(row0, ROWS_PER_TILE)],
                    out_hbm.at[core, pl.ds(row0, ROWS_PER_TILE)])


_edge_pass = functools.partial(
    pl.kernel,
    out_type=jax.ShapeDtypeStruct((2, NP, WIDE), jnp.float32),
    mesh=plsc.VectorSubcoreMesh(core_axis_name="c", subcore_axis_name="s"),
    compiler_params=pltpu.CompilerParams(needs_layout_passes=False,
                                         use_tc_tiling_on_sc=False),
    scratch_types=[
        pltpu.VMEM((NBLK_ALL, BLK), jnp.int32),
        pltpu.VMEM((NBLK_ALL, BLK), jnp.int32),
        pltpu.VMEM((NP,), jnp.float32),
        pltpu.VMEM((NP,), jnp.float32),
        pltpu.VMEM((BLK, WIDE), jnp.float32),
        pltpu.VMEM((BLK, WIDE), jnp.float32),
        pltpu.VMEM((BLK + 16,), jnp.float32),
        pltpu.VMEM_SHARED((NP, WIDE), jnp.float32),
        pltpu.SemaphoreType.DMA,
        pltpu.SemaphoreType.DMA,
        pltpu.SemaphoreType.DMA,
        pltpu.SemaphoreType.DMA,
    ],
)(_edge_pass_body)


def _widen(w, a_src, a_dst):
    k = w.shape[0]
    m = jnp.zeros((k, WIDE), jnp.float32).at[:, :H].set(w)
    wsd = (jnp.zeros((8, k), jnp.float32)
           .at[0].set(jnp.dot(w, a_src, precision=_PREC))
           .at[1].set(jnp.dot(w, a_dst, precision=_PREC)))
    return m, wsd


def kernel(x, edge_index, W1, a_src1, a_dst1, b1, W2, a_src2, a_dst2, b2,
           Wf, bf):
    m1, wsd1 = _widen(W1, a_src1, a_dst1)
    m2, wsd2 = _widen(W2, a_src2, a_dst2)
    b1r = b1.reshape(1, H)
    b2r = b2.reshape(1, H)
    bfr = bf.reshape(1, H)

    loops = jnp.arange(N, dtype=jnp.int32)
    npad = E_PAD - (edge_index.shape[1] + N)
    src = jnp.concatenate([edge_index[0].astype(jnp.int32), loops,
                           jnp.zeros((npad,), jnp.int32)])
    dst = jnp.concatenate([edge_index[1].astype(jnp.int32), loops,
                           jnp.full((npad,), DUMMY, jnp.int32)])
    src3 = jnp.concatenate(
        [src.reshape(NTILES, NBLK, BLK),
         jnp.zeros((NTILES, 2, BLK), jnp.int32)], axis=1)
    dst3 = jnp.concatenate(
        [dst.reshape(NTILES, NBLK, BLK),
         jnp.full((NTILES, 2, BLK), DUMMY, jnp.int32)], axis=1)

    x_pad = jnp.zeros((NP, D_IN), jnp.float32).at[:N].set(x)
    zeros_blk = jnp.zeros((ROWS_PER_TILE, WIDE), jnp.float32)

    h1, sdt1 = _tc_first(x_pad, m1, wsd1)
    acc1 = _edge_pass(h1, sdt1, src3, dst3, zeros_blk)
    h2, sdt2 = _tc_mid(acc1, b1r, m2, wsd2)
    acc2 = _edge_pass(h2, sdt2, src3, dst3, zeros_blk)
    out = _tc_final(acc2, b2r, Wf, bfr)
    return out[:N]


# speedup vs baseline: 1.9216x; 1.5662x over previous
import functools

import jax
import jax.numpy as jnp
from jax import lax
from jax.experimental import pallas as pl
from jax.experimental.pallas import tpu as pltpu
from jax.experimental.pallas import tpu_sc as plsc

N = 10000
D_IN = 128
H = 32
WIDE = 48
DUMMY = N
NP = 10240
BM = 512
NTILES = 32
BLK = 128
NBLK = 82
NBLK_ALL = NBLK + 2
EPT = NBLK * BLK
E_PAD = NTILES * EPT
ROWS_PER_TILE = NP // 16

_PREC = jax.lax.Precision.HIGHEST


def _tc_first_body(x_ref, m_ref, wsd_ref, h_ref, sdt_ref):
    xb = x_ref[...]
    ones_col = (jax.lax.broadcasted_iota(jnp.int32, (1, WIDE), 1) == H)
    h_ref[...] = (jnp.dot(xb, m_ref[...], precision=_PREC,
                          preferred_element_type=jnp.float32)
                  + ones_col.astype(jnp.float32))
    sdt_ref[...] = jax.lax.dot_general(
        wsd_ref[...], xb, (((1,), (1,)), ((), ())),
        precision=_PREC, preferred_element_type=jnp.float32)


def _tc_mid_body(acc_ref, b_ref, m_ref, wsd_ref, h_ref, sdt_ref):
    a = acc_ref[...]
    num = a[0, :, :H] + a[1, :, :H]
    den = a[0, :, H:H + 1] + a[1, :, H:H + 1]
    hprev = jnp.maximum(num / (den + 1e-16) + b_ref[...], 0.0)
    ones_col = (jax.lax.broadcasted_iota(jnp.int32, (1, WIDE), 1) == H)
    h_ref[...] = (jnp.dot(hprev, m_ref[...], precision=_PREC,
                          preferred_element_type=jnp.float32)
                  + ones_col.astype(jnp.float32))
    sdt_ref[...] = jax.lax.dot_general(
        wsd_ref[...], hprev, (((1,), (1,)), ((), ())),
        precision=_PREC, preferred_element_type=jnp.float32)


def _tc_final_body(acc_ref, b_ref, wf_ref, bf_ref, out_ref):
    a = acc_ref[...]
    num = a[0, :, :H] + a[1, :, :H]
    den = a[0, :, H:H + 1] + a[1, :, H:H + 1]
    hprev = jnp.maximum(num / (den + 1e-16) + b_ref[...], 0.0)
    out_ref[...] = (jnp.dot(hprev, wf_ref[...], precision=_PREC,
                            preferred_element_type=jnp.float32)
                    + bf_ref[...])


_G = NP // BM


def _tc_first(x_pad, m1, wsd1):
    return pl.pallas_call(
        _tc_first_body,
        grid=(_G,),
        in_specs=[
            pl.BlockSpec((BM, D_IN), lambda i: (i, 0)),
            pl.BlockSpec((D_IN, WIDE), lambda i: (0, 0)),
            pl.BlockSpec((8, D_IN), lambda i: (0, 0)),
        ],
        out_specs=[
            pl.BlockSpec((BM, WIDE), lambda i: (i, 0)),
            pl.BlockSpec((8, BM), lambda i: (0, i)),
        ],
        out_shape=[
            jax.ShapeDtypeStruct((NP, WIDE), jnp.float32),
            jax.ShapeDtypeStruct((8, NP), jnp.float32),
        ],
    )(x_pad, m1, wsd1)


def _tc_mid(acc, bvec, m2, wsd2):
    return pl.pallas_call(
        _tc_mid_body,
        grid=(_G,),
        in_specs=[
            pl.BlockSpec((2, BM, WIDE), lambda i: (0, i, 0)),
            pl.BlockSpec((1, H), lambda i: (0, 0)),
            pl.BlockSpec((H, WIDE), lambda i: (0, 0)),
            pl.BlockSpec((8, H), lambda i: (0, 0)),
        ],
        out_specs=[
            pl.BlockSpec((BM, WIDE), lambda i: (i, 0)),
            pl.BlockSpec((8, BM), lambda i: (0, i)),
        ],
        out_shape=[
            jax.ShapeDtypeStruct((NP, WIDE), jnp.float32),
            jax.ShapeDtypeStruct((8, NP), jnp.float32),
        ],
    )(acc, bvec, m2, wsd2)


def _tc_final(acc, bvec, wf, bf):
    return pl.pallas_call(
        _tc_final_body,
        grid=(_G,),
        in_specs=[
            pl.BlockSpec((2, BM, WIDE), lambda i: (0, i, 0)),
            pl.BlockSpec((1, H), lambda i: (0, 0)),
            pl.BlockSpec((H, H), lambda i: (0, 0)),
            pl.BlockSpec((1, H), lambda i: (0, 0)),
        ],
        out_specs=pl.BlockSpec((BM, H), lambda i: (i, 0)),
        out_shape=jax.ShapeDtypeStruct((NP, H), jnp.float32),
    )(acc, bvec, wf, bf)


def _edge_pass_body(h_hbm, sdt_hbm, src_hbm, dst_hbm, zeros_hbm, out_hbm,
                    src_v, dst_v, s_v, d_v, msg_a, msg_b, ex_v, acc_sh, h_sh,
                    gsem_a, gsem_b, ssem_a, ssem_b):
    core = lax.axis_index("c")
    sub = lax.axis_index("s")
    wid = sub * 2 + core

    pltpu.sync_copy(src_hbm.at[wid], src_v)
    pltpu.sync_copy(dst_hbm.at[wid], dst_v)
    pltpu.sync_copy(sdt_hbm.at[0], s_v)
    pltpu.sync_copy(sdt_hbm.at[1], d_v)
    pltpu.sync_copy(zeros_hbm, acc_sh.at[pl.ds(sub * ROWS_PER_TILE,
                                               ROWS_PER_TILE)])
    pltpu.sync_copy(h_hbm.at[pl.ds(sub * ROWS_PER_TILE, ROWS_PER_TILE)],
                    h_sh.at[pl.ds(sub * ROWS_PER_TILE, ROWS_PER_TILE)])
    plsc.subcore_barrier()

    def alpha(b):
        for g in range(BLK // 16):
            src16 = src_v[b, pl.ds(g * 16, 16)]
            dst16 = dst_v[b, pl.ds(g * 16, 16)]
            al = plsc.load_gather(s_v, [src16]) + plsc.load_gather(d_v, [dst16])
            al = jnp.maximum(al, 0.2 * al)
            ex_v[pl.ds(16 + g * 16, 16)] = jnp.exp(al)

    def scale(msg_v):
        for j in range(BLK):
            e = plsc.load_gather(ex_v, [jnp.full((16,), 16 + j, jnp.int32)])
            msg_v[j, pl.ds(0, 16)] = msg_v[j, pl.ds(0, 16)] * e
            msg_v[j, pl.ds(16, 16)] = msg_v[j, pl.ds(16, 16)] * e
            msg_v[j, pl.ds(32, 16)] = e

    def gather(b, msg_v, sem):
        return pltpu.async_copy(h_sh.at[src_v.at[b]], msg_v, sem)

    def body(b, carry):
        cp = gather(b, msg_a, gsem_a)
        alpha(b)
        cp.wait()
        scale(msg_a)
        pltpu.sync_copy(msg_a, acc_sh.at[dst_v.at[b]], add=True)
        return carry

    lax.fori_loop(0, NBLK, body, 0)

    plsc.subcore_barrier()
    row0 = sub * ROWS_PER_TILE
    pltpu.sync_copy(acc_sh.at[pl.ds(row0, ROWS_PER_TILE)],
                    out_hbm.at[core, pl.ds(row0, ROWS_PER_TILE)])


_edge_pass = functools.partial(
    pl.kernel,
    out_type=jax.ShapeDtypeStruct((2, NP, WIDE), jnp.float32),
    mesh=plsc.VectorSubcoreMesh(core_axis_name="c", subcore_axis_name="s"),
    compiler_params=pltpu.CompilerParams(needs_layout_passes=False,
                                         use_tc_tiling_on_sc=False),
    scratch_types=[
        pltpu.VMEM((NBLK_ALL, BLK), jnp.int32),
        pltpu.VMEM((NBLK_ALL, BLK), jnp.int32),
        pltpu.VMEM((NP,), jnp.float32),
        pltpu.VMEM((NP,), jnp.float32),
        pltpu.VMEM((BLK, WIDE), jnp.float32),
        pltpu.VMEM((BLK, WIDE), jnp.float32),
        pltpu.VMEM((BLK + 16,), jnp.float32),
        pltpu.VMEM_SHARED((NP, WIDE), jnp.float32),
        pltpu.VMEM_SHARED((NP, WIDE), jnp.float32),
        pltpu.SemaphoreType.DMA,
        pltpu.SemaphoreType.DMA,
        pltpu.SemaphoreType.DMA,
        pltpu.SemaphoreType.DMA,
    ],
)(_edge_pass_body)


def _widen(w, a_src, a_dst):
    k = w.shape[0]
    m = jnp.zeros((k, WIDE), jnp.float32).at[:, :H].set(w)
    wsd = (jnp.zeros((8, k), jnp.float32)
           .at[0].set(jnp.dot(w, a_src, precision=_PREC))
           .at[1].set(jnp.dot(w, a_dst, precision=_PREC)))
    return m, wsd


def kernel(x, edge_index, W1, a_src1, a_dst1, b1, W2, a_src2, a_dst2, b2,
           Wf, bf):
    m1, wsd1 = _widen(W1, a_src1, a_dst1)
    m2, wsd2 = _widen(W2, a_src2, a_dst2)
    b1r = b1.reshape(1, H)
    b2r = b2.reshape(1, H)
    bfr = bf.reshape(1, H)

    loops = jnp.arange(N, dtype=jnp.int32)
    npad = E_PAD - (edge_index.shape[1] + N)
    src = jnp.concatenate([edge_index[0].astype(jnp.int32), loops,
                           jnp.zeros((npad,), jnp.int32)])
    dst = jnp.concatenate([edge_index[1].astype(jnp.int32), loops,
                           jnp.full((npad,), DUMMY, jnp.int32)])
    src3 = jnp.concatenate(
        [src.reshape(NTILES, NBLK, BLK),
         jnp.zeros((NTILES, 2, BLK), jnp.int32)], axis=1)
    dst3 = jnp.concatenate(
        [dst.reshape(NTILES, NBLK, BLK),
         jnp.full((NTILES, 2, BLK), DUMMY, jnp.int32)], axis=1)

    x_pad = jnp.zeros((NP, D_IN), jnp.float32).at[:N].set(x)
    zeros_blk = jnp.zeros((ROWS_PER_TILE, WIDE), jnp.float32)

    h1, sdt1 = _tc_first(x_pad, m1, wsd1)
    acc1 = _edge_pass(h1, sdt1, src3, dst3, zeros_blk)
    h2, sdt2 = _tc_mid(acc1, b1r, m2, wsd2)
    acc2 = _edge_pass(h2, sdt2, src3, dst3, zeros_blk)
    out = _tc_final(acc2, b2r, Wf, bfr)
    return out[:N]


# speedup vs baseline: 2.3059x; 1.2000x over previous
import functools

import jax
import jax.numpy as jnp
from jax import lax
from jax.experimental import pallas as pl
from jax.experimental.pallas import tpu as pltpu
from jax.experimental.pallas import tpu_sc as plsc

N = 10000
D_IN = 128
H = 32
WIDE = 48
DUMMY = N
NP = 10240
BM = 512
NTILES = 32
BLK = 128
NBLK = 82
NBLK_ALL = NBLK + 2
EPT = NBLK * BLK
E_PAD = NTILES * EPT
ROWS_PER_TILE = NP // 16

_PREC = jax.lax.Precision.HIGHEST


def _tc_first_body(x_ref, m_ref, wsd_ref, h_ref, sdt_ref):
    xb = x_ref[...]
    ones_col = (jax.lax.broadcasted_iota(jnp.int32, (1, WIDE), 1) == H)
    h_ref[...] = (jnp.dot(xb, m_ref[...], precision=_PREC,
                          preferred_element_type=jnp.float32)
                  + ones_col.astype(jnp.float32))
    sdt_ref[...] = jax.lax.dot_general(
        wsd_ref[...], xb, (((1,), (1,)), ((), ())),
        precision=_PREC, preferred_element_type=jnp.float32)


def _tc_mid_body(acc_ref, b_ref, m_ref, wsd_ref, h_ref, sdt_ref):
    a = acc_ref[...]
    num = a[0, :, :H] + a[1, :, :H]
    den = a[0, :, H:H + 1] + a[1, :, H:H + 1]
    hprev = jnp.maximum(num / (den + 1e-16) + b_ref[...], 0.0)
    ones_col = (jax.lax.broadcasted_iota(jnp.int32, (1, WIDE), 1) == H)
    h_ref[...] = (jnp.dot(hprev, m_ref[...], precision=_PREC,
                          preferred_element_type=jnp.float32)
                  + ones_col.astype(jnp.float32))
    sdt_ref[...] = jax.lax.dot_general(
        wsd_ref[...], hprev, (((1,), (1,)), ((), ())),
        precision=_PREC, preferred_element_type=jnp.float32)


def _tc_final_body(acc_ref, b_ref, wf_ref, bf_ref, out_ref):
    a = acc_ref[...]
    num = a[0, :, :H] + a[1, :, :H]
    den = a[0, :, H:H + 1] + a[1, :, H:H + 1]
    hprev = jnp.maximum(num / (den + 1e-16) + b_ref[...], 0.0)
    out_ref[...] = (jnp.dot(hprev, wf_ref[...], precision=_PREC,
                            preferred_element_type=jnp.float32)
                    + bf_ref[...])


_G = NP // BM


def _tc_first(x_pad, m1, wsd1):
    return pl.pallas_call(
        _tc_first_body,
        grid=(_G,),
        in_specs=[
            pl.BlockSpec((BM, D_IN), lambda i: (i, 0)),
            pl.BlockSpec((D_IN, WIDE), lambda i: (0, 0)),
            pl.BlockSpec((8, D_IN), lambda i: (0, 0)),
        ],
        out_specs=[
            pl.BlockSpec((BM, WIDE), lambda i: (i, 0)),
            pl.BlockSpec((8, BM), lambda i: (0, i)),
        ],
        out_shape=[
            jax.ShapeDtypeStruct((NP, WIDE), jnp.float32),
            jax.ShapeDtypeStruct((8, NP), jnp.float32),
        ],
    )(x_pad, m1, wsd1)


def _tc_mid(acc, bvec, m2, wsd2):
    return pl.pallas_call(
        _tc_mid_body,
        grid=(_G,),
        in_specs=[
            pl.BlockSpec((2, BM, WIDE), lambda i: (0, i, 0)),
            pl.BlockSpec((1, H), lambda i: (0, 0)),
            pl.BlockSpec((H, WIDE), lambda i: (0, 0)),
            pl.BlockSpec((8, H), lambda i: (0, 0)),
        ],
        out_specs=[
            pl.BlockSpec((BM, WIDE), lambda i: (i, 0)),
            pl.BlockSpec((8, BM), lambda i: (0, i)),
        ],
        out_shape=[
            jax.ShapeDtypeStruct((NP, WIDE), jnp.float32),
            jax.ShapeDtypeStruct((8, NP), jnp.float32),
        ],
    )(acc, bvec, m2, wsd2)


def _tc_final(acc, bvec, wf, bf):
    return pl.pallas_call(
        _tc_final_body,
        grid=(_G,),
        in_specs=[
            pl.BlockSpec((2, BM, WIDE), lambda i: (0, i, 0)),
            pl.BlockSpec((1, H), lambda i: (0, 0)),
            pl.BlockSpec((H, H), lambda i: (0, 0)),
            pl.BlockSpec((1, H), lambda i: (0, 0)),
        ],
        out_specs=pl.BlockSpec((BM, H), lambda i: (i, 0)),
        out_shape=jax.ShapeDtypeStruct((NP, H), jnp.float32),
    )(acc, bvec, wf, bf)


def _edge_pass_body(h_hbm, sdt_hbm, src_hbm, dst_hbm, zeros_hbm, out_hbm,
                    src_v, dst_v, s_v, d_v, msg_a, msg_b, ex_v, acc_sh, h_sh,
                    gsem_a, gsem_b, ssem_a, ssem_b):
    core = lax.axis_index("c")
    sub = lax.axis_index("s")
    wid = sub * 2 + core

    pltpu.sync_copy(src_hbm.at[wid], src_v)
    pltpu.sync_copy(dst_hbm.at[wid], dst_v)
    pltpu.sync_copy(sdt_hbm.at[0], s_v)
    pltpu.sync_copy(sdt_hbm.at[1], d_v)
    pltpu.sync_copy(zeros_hbm, acc_sh.at[pl.ds(sub * ROWS_PER_TILE,
                                               ROWS_PER_TILE)])
    pltpu.sync_copy(h_hbm.at[pl.ds(sub * ROWS_PER_TILE, ROWS_PER_TILE)],
                    h_sh.at[pl.ds(sub * ROWS_PER_TILE, ROWS_PER_TILE)])
    plsc.subcore_barrier()

    def alpha(b):
        @plsc.parallel_loop(0, BLK, 16, unroll=4)
        def _(g0):
            src16 = src_v[b, pl.ds(g0, 16)]
            dst16 = dst_v[b, pl.ds(g0, 16)]
            al = plsc.load_gather(s_v, [src16]) + plsc.load_gather(d_v, [dst16])
            al = jnp.maximum(al, 0.2 * al)
            ex_v[pl.ds(16 + g0, 16)] = jnp.exp(al)

    def scale(msg_v):
        @plsc.parallel_loop(0, BLK, 1, unroll=8)
        def _(j):
            e = plsc.load_gather(ex_v, [jnp.full((16,), 16, jnp.int32) + j])
            msg_v[j, pl.ds(0, 16)] = msg_v[j, pl.ds(0, 16)] * e
            msg_v[j, pl.ds(16, 16)] = msg_v[j, pl.ds(16, 16)] * e
            msg_v[j, pl.ds(32, 16)] = e

    def gather(b, msg_v, sem):
        return pltpu.async_copy(h_sh.at[src_v.at[b]], msg_v, sem)

    def body(b, carry):
        cp = gather(b, msg_a, gsem_a)
        alpha(b)
        cp.wait()
        scale(msg_a)
        pltpu.sync_copy(msg_a, acc_sh.at[dst_v.at[b]], add=True)
        return carry

    lax.fori_loop(0, NBLK, body, 0)

    plsc.subcore_barrier()
    row0 = sub * ROWS_PER_TILE
    pltpu.sync_copy(acc_sh.at[pl.ds(row0, ROWS_PER_TILE)],
                    out_hbm.at[core, pl.ds(row0, ROWS_PER_TILE)])


_edge_pass = functools.partial(
    pl.kernel,
    out_type=jax.ShapeDtypeStruct((2, NP, WIDE), jnp.float32),
    mesh=plsc.VectorSubcoreMesh(core_axis_name="c", subcore_axis_name="s"),
    compiler_params=pltpu.CompilerParams(needs_layout_passes=False,
                                         use_tc_tiling_on_sc=False),
    scratch_types=[
        pltpu.VMEM((NBLK_ALL, BLK), jnp.int32),
        pltpu.VMEM((NBLK_ALL, BLK), jnp.int32),
        pltpu.VMEM((NP,), jnp.float32),
        pltpu.VMEM((NP,), jnp.float32),
        pltpu.VMEM((BLK, WIDE), jnp.float32),
        pltpu.VMEM((BLK, WIDE), jnp.float32),
        pltpu.VMEM((BLK + 16,), jnp.float32),
        pltpu.VMEM_SHARED((NP, WIDE), jnp.float32),
        pltpu.VMEM_SHARED((NP, WIDE), jnp.float32),
        pltpu.SemaphoreType.DMA,
        pltpu.SemaphoreType.DMA,
        pltpu.SemaphoreType.DMA,
        pltpu.SemaphoreType.DMA,
    ],
)(_edge_pass_body)


def _widen(w, a_src, a_dst):
    k = w.shape[0]
    m = jnp.zeros((k, WIDE), jnp.float32).at[:, :H].set(w)
    wsd = (jnp.zeros((8, k), jnp.float32)
           .at[0].set(jnp.dot(w, a_src, precision=_PREC))
           .at[1].set(jnp.dot(w, a_dst, precision=_PREC)))
    return m, wsd


def kernel(x, edge_index, W1, a_src1, a_dst1, b1, W2, a_src2, a_dst2, b2,
           Wf, bf):
    m1, wsd1 = _widen(W1, a_src1, a_dst1)
    m2, wsd2 = _widen(W2, a_src2, a_dst2)
    b1r = b1.reshape(1, H)
    b2r = b2.reshape(1, H)
    bfr = bf.reshape(1, H)

    loops = jnp.arange(N, dtype=jnp.int32)
    npad = E_PAD - (edge_index.shape[1] + N)
    src = jnp.concatenate([edge_index[0].astype(jnp.int32), loops,
                           jnp.zeros((npad,), jnp.int32)])
    dst = jnp.concatenate([edge_index[1].astype(jnp.int32), loops,
                           jnp.full((npad,), DUMMY, jnp.int32)])
    src3 = jnp.concatenate(
        [src.reshape(NTILES, NBLK, BLK),
         jnp.zeros((NTILES, 2, BLK), jnp.int32)], axis=1)
    dst3 = jnp.concatenate(
        [dst.reshape(NTILES, NBLK, BLK),
         jnp.full((NTILES, 2, BLK), DUMMY, jnp.int32)], axis=1)

    x_pad = jnp.zeros((NP, D_IN), jnp.float32).at[:N].set(x)
    zeros_blk = jnp.zeros((ROWS_PER_TILE, WIDE), jnp.float32)

    h1, sdt1 = _tc_first(x_pad, m1, wsd1)
    acc1 = _edge_pass(h1, sdt1, src3, dst3, zeros_blk)
    h2, sdt2 = _tc_mid(acc1, b1r, m2, wsd2)
    acc2 = _edge_pass(h2, sdt2, src3, dst3, zeros_blk)
    out = _tc_final(acc2, b2r, Wf, bfr)
    return out[:N]


# speedup vs baseline: 2.6691x; 1.1575x over previous
import functools

import jax
import jax.numpy as jnp
from jax import lax
from jax.experimental import pallas as pl
from jax.experimental.pallas import tpu as pltpu
from jax.experimental.pallas import tpu_sc as plsc

N = 10000
D_IN = 128
H = 32
DUMMY = N
NP = 10240
BM = 512
NTILES = 32
BLK = 128
NBLK = 82
NBLK_ALL = NBLK + 2
EPT = NBLK * BLK
E_PAD = NTILES * EPT
ROWS_PER_TILE = NP // 16

_PREC = jax.lax.Precision.HIGHEST


def _tc_first_body(x_ref, m_ref, wsd_ref, h_ref, sdt_ref):
    xb = x_ref[...]
    h_ref[...] = jnp.dot(xb, m_ref[...], precision=_PREC,
                         preferred_element_type=jnp.float32)
    sdt_ref[...] = jax.lax.dot_general(
        wsd_ref[...], xb, (((1,), (1,)), ((), ())),
        precision=_PREC, preferred_element_type=jnp.float32)


def _combine(acc_ref, den_ref, b_ref):
    a = acc_ref[...]
    num = a[0] + a[1]
    d = den_ref[...]
    den = jnp.reshape(jnp.sum(d, axis=(0, 1)), (BM, 1))
    return jnp.maximum(num / (den + 1e-16) + b_ref[...], 0.0)


def _tc_mid_body(acc_ref, den_ref, b_ref, m_ref, wsd_ref, h_ref, sdt_ref):
    hprev = _combine(acc_ref, den_ref, b_ref)
    h_ref[...] = jnp.dot(hprev, m_ref[...], precision=_PREC,
                         preferred_element_type=jnp.float32)
    sdt_ref[...] = jax.lax.dot_general(
        wsd_ref[...], hprev, (((1,), (1,)), ((), ())),
        precision=_PREC, preferred_element_type=jnp.float32)


def _tc_final_body(acc_ref, den_ref, b_ref, wf_ref, bf_ref, out_ref):
    hprev = _combine(acc_ref, den_ref, b_ref)
    out_ref[...] = (jnp.dot(hprev, wf_ref[...], precision=_PREC,
                            preferred_element_type=jnp.float32)
                    + bf_ref[...])


_G = NP // BM


def _tc_first(x_pad, m1, wsd1):
    return pl.pallas_call(
        _tc_first_body,
        grid=(_G,),
        in_specs=[
            pl.BlockSpec((BM, D_IN), lambda i: (i, 0)),
            pl.BlockSpec((D_IN, H), lambda i: (0, 0)),
            pl.BlockSpec((8, D_IN), lambda i: (0, 0)),
        ],
        out_specs=[
            pl.BlockSpec((BM, H), lambda i: (i, 0)),
            pl.BlockSpec((8, BM), lambda i: (0, i)),
        ],
        out_shape=[
            jax.ShapeDtypeStruct((NP, H), jnp.float32),
            jax.ShapeDtypeStruct((8, NP), jnp.float32),
        ],
    )(x_pad, m1, wsd1)


def _tc_mid(acc, den, bvec, m2, wsd2):
    return pl.pallas_call(
        _tc_mid_body,
        grid=(_G,),
        in_specs=[
            pl.BlockSpec((2, BM, H), lambda i: (0, i, 0)),
            pl.BlockSpec((2, 16, BM), lambda i: (0, 0, i)),
            pl.BlockSpec((1, H), lambda i: (0, 0)),
            pl.BlockSpec((H, H), lambda i: (0, 0)),
            pl.BlockSpec((8, H), lambda i: (0, 0)),
        ],
        out_specs=[
            pl.BlockSpec((BM, H), lambda i: (i, 0)),
            pl.BlockSpec((8, BM), lambda i: (0, i)),
        ],
        out_shape=[
            jax.ShapeDtypeStruct((NP, H), jnp.float32),
            jax.ShapeDtypeStruct((8, NP), jnp.float32),
        ],
    )(acc, den, bvec, m2, wsd2)


def _tc_final(acc, den, bvec, wf, bf):
    return pl.pallas_call(
        _tc_final_body,
        grid=(_G,),
        in_specs=[
            pl.BlockSpec((2, BM, H), lambda i: (0, i, 0)),
            pl.BlockSpec((2, 16, BM), lambda i: (0, 0, i)),
            pl.BlockSpec((1, H), lambda i: (0, 0)),
            pl.BlockSpec((H, H), lambda i: (0, 0)),
            pl.BlockSpec((1, H), lambda i: (0, 0)),
        ],
        out_specs=pl.BlockSpec((BM, H), lambda i: (i, 0)),
        out_shape=jax.ShapeDtypeStruct((NP, H), jnp.float32),
    )(acc, den, bvec, wf, bf)


def _edge_pass_body(h_hbm, sdt_hbm, src_hbm, dst_hbm, zeros_hbm,
                    acc_out, den_out,
                    src_v, dst_v, s_v, d_v, msg_a, ex_v, den_l,
                    acc_sh, h_sh, gsem_a):
    core = lax.axis_index("c")
    sub = lax.axis_index("s")
    wid = sub * 2 + core
    row0 = sub * ROWS_PER_TILE

    pltpu.sync_copy(src_hbm.at[wid], src_v)
    pltpu.sync_copy(dst_hbm.at[wid], dst_v)
    pltpu.sync_copy(sdt_hbm.at[0], s_v)
    pltpu.sync_copy(sdt_hbm.at[1], d_v)

    @plsc.parallel_loop(0, NP, 16, unroll=8)
    def _(i):
        den_l[pl.ds(i, 16)] = jnp.zeros((16,), jnp.float32)

    pltpu.sync_copy(zeros_hbm, acc_sh.at[pl.ds(row0, ROWS_PER_TILE)])
    pltpu.sync_copy(h_hbm.at[pl.ds(row0, ROWS_PER_TILE)],
                    h_sh.at[pl.ds(row0, ROWS_PER_TILE)])
    plsc.subcore_barrier()

    def alpha(b):
        @plsc.parallel_loop(0, BLK, 16, unroll=4)
        def _(g0):
            src16 = src_v[b, pl.ds(g0, 16)]
            dst16 = dst_v[b, pl.ds(g0, 16)]
            al = plsc.load_gather(s_v, [src16]) + plsc.load_gather(d_v, [dst16])
            al = jnp.maximum(al, 0.2 * al)
            ex_v[pl.ds(16 + g0, 16)] = jnp.exp(al)
        for g in range(BLK // 16):
            dst16 = dst_v[b, pl.ds(g * 16, 16)]
            ex16 = ex_v[pl.ds(16 + g * 16, 16)]
            plsc.addupdate_scatter(den_l, [dst16], ex16)

    def scale(msg_v):
        @plsc.parallel_loop(0, BLK, 1, unroll=8)
        def _(j):
            e = plsc.load_gather(ex_v, [jnp.full((16,), 16, jnp.int32) + j])
            msg_v[j, pl.ds(0, 16)] = msg_v[j, pl.ds(0, 16)] * e
            msg_v[j, pl.ds(16, 16)] = msg_v[j, pl.ds(16, 16)] * e

    def body(b, carry):
        cp = pltpu.async_copy(h_sh.at[src_v.at[b]], msg_a, gsem_a)
        alpha(b)
        cp.wait()
        scale(msg_a)
        pltpu.sync_copy(msg_a, acc_sh.at[dst_v.at[b]], add=True)
        return carry

    lax.fori_loop(0, NBLK, body, 0)

    pltpu.sync_copy(den_l, den_out.at[core, sub])
    plsc.subcore_barrier()
    pltpu.sync_copy(acc_sh.at[pl.ds(row0, ROWS_PER_TILE)],
                    acc_out.at[core, pl.ds(row0, ROWS_PER_TILE)])


_edge_pass = functools.partial(
    pl.kernel,
    out_type=(jax.ShapeDtypeStruct((2, NP, H), jnp.float32),
              jax.ShapeDtypeStruct((2, 16, NP), jnp.float32)),
    mesh=plsc.VectorSubcoreMesh(core_axis_name="c", subcore_axis_name="s"),
    compiler_params=pltpu.CompilerParams(needs_layout_passes=False,
                                         use_tc_tiling_on_sc=False),
    scratch_types=[
        pltpu.VMEM((NBLK_ALL, BLK), jnp.int32),
        pltpu.VMEM((NBLK_ALL, BLK), jnp.int32),
        pltpu.VMEM((NP,), jnp.float32),
        pltpu.VMEM((NP,), jnp.float32),
        pltpu.VMEM((BLK, H), jnp.float32),
        pltpu.VMEM((BLK + 16,), jnp.float32),
        pltpu.VMEM((NP,), jnp.float32),
        pltpu.VMEM_SHARED((NP, H), jnp.float32),
        pltpu.VMEM_SHARED((NP, H), jnp.float32),
        pltpu.SemaphoreType.DMA,
    ],
)(_edge_pass_body)


def _widen(w, a_src, a_dst):
    k = w.shape[0]
    wsd = (jnp.zeros((8, k), jnp.float32)
           .at[0].set(jnp.dot(w, a_src, precision=_PREC))
           .at[1].set(jnp.dot(w, a_dst, precision=_PREC)))
    return wsd


def kernel(x, edge_index, W1, a_src1, a_dst1, b1, W2, a_src2, a_dst2, b2,
           Wf, bf):
    wsd1 = _widen(W1, a_src1, a_dst1)
    wsd2 = _widen(W2, a_src2, a_dst2)
    b1r = b1.reshape(1, H)
    b2r = b2.reshape(1, H)
    bfr = bf.reshape(1, H)

    loops = jnp.arange(N, dtype=jnp.int32)
    npad = E_PAD - (edge_index.shape[1] + N)
    src = jnp.concatenate([edge_index[0].astype(jnp.int32), loops,
                           jnp.zeros((npad,), jnp.int32)])
    dst = jnp.concatenate([edge_index[1].astype(jnp.int32), loops,
                           jnp.full((npad,), DUMMY, jnp.int32)])
    src3 = jnp.concatenate(
        [src.reshape(NTILES, NBLK, BLK),
         jnp.zeros((NTILES, 2, BLK), jnp.int32)], axis=1)
    dst3 = jnp.concatenate(
        [dst.reshape(NTILES, NBLK, BLK),
         jnp.full((NTILES, 2, BLK), DUMMY, jnp.int32)], axis=1)

    x_pad = jnp.zeros((NP, D_IN), jnp.float32).at[:N].set(x)
    zeros_blk = jnp.zeros((ROWS_PER_TILE, H), jnp.float32)

    h1, sdt1 = _tc_first(x_pad, W1, wsd1)
    acc1, den1 = _edge_pass(h1, sdt1, src3, dst3, zeros_blk)
    h2, sdt2 = _tc_mid(acc1, den1, b1r, W2, wsd2)
    acc2, den2 = _edge_pass(h2, sdt2, src3, dst3, zeros_blk)
    out = _tc_final(acc2, den2, b2r, Wf, bfr)
    return out[:N]


# speedup vs baseline: 2.6694x; 1.0001x over previous
import functools

import jax
import jax.numpy as jnp
from jax import lax
from jax.experimental import pallas as pl
from jax.experimental.pallas import tpu as pltpu
from jax.experimental.pallas import tpu_sc as plsc

N = 10000
D_IN = 128
H = 32
DUMMY = N
NP = 10240
BM = 512
NTILES = 32
BLK = 128
NBLK = 82
NBLK_ALL = NBLK + 2
EPT = NBLK * BLK
E_PAD = NTILES * EPT
ROWS_PER_TILE = NP // 16

_PREC = jax.lax.Precision.HIGHEST


def _tc_first_body(x_ref, m_ref, wsd_ref, h_ref, sdt_ref):
    xb = x_ref[...]
    h_ref[...] = jnp.dot(xb, m_ref[...], precision=_PREC,
                         preferred_element_type=jnp.float32)
    sdt_ref[...] = jax.lax.dot_general(
        wsd_ref[...], xb, (((1,), (1,)), ((), ())),
        precision=_PREC, preferred_element_type=jnp.float32)


def _combine(acc_ref, den_ref, b_ref):
    a = acc_ref[...]
    num = a[0] + a[1]
    d = den_ref[...]
    den = jnp.reshape(jnp.sum(d, axis=(0, 1)), (BM, 1))
    return jnp.maximum(num / (den + 1e-16) + b_ref[...], 0.0)


def _tc_mid_body(acc_ref, den_ref, b_ref, m_ref, wsd_ref, h_ref, sdt_ref):
    hprev = _combine(acc_ref, den_ref, b_ref)
    h_ref[...] = jnp.dot(hprev, m_ref[...], precision=_PREC,
                         preferred_element_type=jnp.float32)
    sdt_ref[...] = jax.lax.dot_general(
        wsd_ref[...], hprev, (((1,), (1,)), ((), ())),
        precision=_PREC, preferred_element_type=jnp.float32)


def _tc_final_body(acc_ref, den_ref, b_ref, wf_ref, bf_ref, out_ref):
    hprev = _combine(acc_ref, den_ref, b_ref)
    out_ref[...] = (jnp.dot(hprev, wf_ref[...], precision=_PREC,
                            preferred_element_type=jnp.float32)
                    + bf_ref[...])


_G = NP // BM


def _tc_first(x_pad, m1, wsd1):
    return pl.pallas_call(
        _tc_first_body,
        grid=(_G,),
        in_specs=[
            pl.BlockSpec((BM, D_IN), lambda i: (i, 0)),
            pl.BlockSpec((D_IN, H), lambda i: (0, 0)),
            pl.BlockSpec((8, D_IN), lambda i: (0, 0)),
        ],
        out_specs=[
            pl.BlockSpec((BM, H), lambda i: (i, 0)),
            pl.BlockSpec((8, BM), lambda i: (0, i)),
        ],
        out_shape=[
            jax.ShapeDtypeStruct((NP, H), jnp.float32),
            jax.ShapeDtypeStruct((8, NP), jnp.float32),
        ],
    )(x_pad, m1, wsd1)


def _tc_mid(acc, den, bvec, m2, wsd2):
    return pl.pallas_call(
        _tc_mid_body,
        grid=(_G,),
        in_specs=[
            pl.BlockSpec((2, BM, H), lambda i: (0, i, 0)),
            pl.BlockSpec((2, 16, BM), lambda i: (0, 0, i)),
            pl.BlockSpec((1, H), lambda i: (0, 0)),
            pl.BlockSpec((H, H), lambda i: (0, 0)),
            pl.BlockSpec((8, H), lambda i: (0, 0)),
        ],
        out_specs=[
            pl.BlockSpec((BM, H), lambda i: (i, 0)),
            pl.BlockSpec((8, BM), lambda i: (0, i)),
        ],
        out_shape=[
            jax.ShapeDtypeStruct((NP, H), jnp.float32),
            jax.ShapeDtypeStruct((8, NP), jnp.float32),
        ],
    )(acc, den, bvec, m2, wsd2)


def _tc_final(acc, den, bvec, wf, bf):
    return pl.pallas_call(
        _tc_final_body,
        grid=(_G,),
        in_specs=[
            pl.BlockSpec((2, BM, H), lambda i: (0, i, 0)),
            pl.BlockSpec((2, 16, BM), lambda i: (0, 0, i)),
            pl.BlockSpec((1, H), lambda i: (0, 0)),
            pl.BlockSpec((H, H), lambda i: (0, 0)),
            pl.BlockSpec((1, H), lambda i: (0, 0)),
        ],
        out_specs=pl.BlockSpec((BM, H), lambda i: (i, 0)),
        out_shape=jax.ShapeDtypeStruct((NP, H), jnp.float32),
    )(acc, den, bvec, wf, bf)


def _edge_pass_body(h_hbm, sdt_hbm, src_hbm, dst_hbm, zeros_hbm,
                    acc_out, den_out,
                    src_v, dst_v, s_v, d_v, msg_a, ex_v, den_l,
                    acc_sh, h_sh, gsem_a):
    core = lax.axis_index("c")
    sub = lax.axis_index("s")
    wid = sub * 2 + core
    row0 = sub * ROWS_PER_TILE

    pltpu.sync_copy(src_hbm.at[wid], src_v)
    pltpu.sync_copy(dst_hbm.at[wid], dst_v)
    pltpu.sync_copy(sdt_hbm.at[0], s_v)
    pltpu.sync_copy(sdt_hbm.at[1], d_v)

    @plsc.parallel_loop(0, NP, 16, unroll=8)
    def _(i):
        den_l[pl.ds(i, 16)] = jnp.zeros((16,), jnp.float32)

    pltpu.sync_copy(zeros_hbm, acc_sh.at[pl.ds(row0, ROWS_PER_TILE)])
    pltpu.sync_copy(h_hbm.at[pl.ds(row0, ROWS_PER_TILE)],
                    h_sh.at[pl.ds(row0, ROWS_PER_TILE)])
    plsc.subcore_barrier()

    def alpha(b):
        @plsc.parallel_loop(0, BLK, 16, unroll=8)
        def _(g0):
            src16 = src_v[b, pl.ds(g0, 16)]
            dst16 = dst_v[b, pl.ds(g0, 16)]
            al = plsc.load_gather(s_v, [src16]) + plsc.load_gather(d_v, [dst16])
            al = jnp.maximum(al, 0.2 * al)
            ex_v[pl.ds(16 + g0, 16)] = jnp.exp(al)
        for g in range(BLK // 16):
            dst16 = dst_v[b, pl.ds(g * 16, 16)]
            ex16 = ex_v[pl.ds(16 + g * 16, 16)]
            plsc.addupdate_scatter(den_l, [dst16], ex16)

    def scale(msg_v):
        @plsc.parallel_loop(0, BLK, 1, unroll=16)
        def _(j):
            e = plsc.load_gather(ex_v, [jnp.full((16,), 16, jnp.int32) + j])
            msg_v[j, pl.ds(0, 16)] = msg_v[j, pl.ds(0, 16)] * e
            msg_v[j, pl.ds(16, 16)] = msg_v[j, pl.ds(16, 16)] * e

    def body(b, carry):
        cp = pltpu.async_copy(h_sh.at[src_v.at[b]], msg_a, gsem_a)
        alpha(b)
        cp.wait()
        scale(msg_a)
        pltpu.sync_copy(msg_a, acc_sh.at[dst_v.at[b]], add=True)
        return carry

    lax.fori_loop(0, NBLK, body, 0)

    pltpu.sync_copy(den_l, den_out.at[core, sub])
    plsc.subcore_barrier()
    pltpu.sync_copy(acc_sh.at[pl.ds(row0, ROWS_PER_TILE)],
                    acc_out.at[core, pl.ds(row0, ROWS_PER_TILE)])


_edge_pass = functools.partial(
    pl.kernel,
    out_type=(jax.ShapeDtypeStruct((2, NP, H), jnp.float32),
              jax.ShapeDtypeStruct((2, 16, NP), jnp.float32)),
    mesh=plsc.VectorSubcoreMesh(core_axis_name="c", subcore_axis_name="s"),
    compiler_params=pltpu.CompilerParams(needs_layout_passes=False,
                                         use_tc_tiling_on_sc=False),
    scratch_types=[
        pltpu.VMEM((NBLK_ALL, BLK), jnp.int32),
        pltpu.VMEM((NBLK_ALL, BLK), jnp.int32),
        pltpu.VMEM((NP,), jnp.float32),
        pltpu.VMEM((NP,), jnp.float32),
        pltpu.VMEM((BLK, H), jnp.float32),
        pltpu.VMEM((BLK + 16,), jnp.float32),
        pltpu.VMEM((NP,), jnp.float32),
        pltpu.VMEM_SHARED((NP, H), jnp.float32),
        pltpu.VMEM_SHARED((NP, H), jnp.float32),
        pltpu.SemaphoreType.DMA,
    ],
)(_edge_pass_body)


def _widen(w, a_src, a_dst):
    k = w.shape[0]
    wsd = (jnp.zeros((8, k), jnp.float32)
           .at[0].set(jnp.dot(w, a_src, precision=_PREC))
           .at[1].set(jnp.dot(w, a_dst, precision=_PREC)))
    return wsd


def kernel(x, edge_index, W1, a_src1, a_dst1, b1, W2, a_src2, a_dst2, b2,
           Wf, bf):
    wsd1 = _widen(W1, a_src1, a_dst1)
    wsd2 = _widen(W2, a_src2, a_dst2)
    b1r = b1.reshape(1, H)
    b2r = b2.reshape(1, H)
    bfr = bf.reshape(1, H)

    loops = jnp.arange(N, dtype=jnp.int32)
    npad = E_PAD - (edge_index.shape[1] + N)
    src = jnp.concatenate([edge_index[0].astype(jnp.int32), loops,
                           jnp.zeros((npad,), jnp.int32)])
    dst = jnp.concatenate([edge_index[1].astype(jnp.int32), loops,
                           jnp.full((npad,), DUMMY, jnp.int32)])
    src3 = jnp.concatenate(
        [src.reshape(NTILES, NBLK, BLK),
         jnp.zeros((NTILES, 2, BLK), jnp.int32)], axis=1)
    dst3 = jnp.concatenate(
        [dst.reshape(NTILES, NBLK, BLK),
         jnp.full((NTILES, 2, BLK), DUMMY, jnp.int32)], axis=1)

    x_pad = jnp.zeros((NP, D_IN), jnp.float32).at[:N].set(x)
    zeros_blk = jnp.zeros((ROWS_PER_TILE, H), jnp.float32)

    h1, sdt1 = _tc_first(x_pad, W1, wsd1)
    acc1, den1 = _edge_pass(h1, sdt1, src3, dst3, zeros_blk)
    h2, sdt2 = _tc_mid(acc1, den1, b1r, W2, wsd2)
    acc2, den2 = _edge_pass(h2, sdt2, src3, dst3, zeros_blk)
    out = _tc_final(acc2, den2, b2r, Wf, bfr)
    return out[:N]
